# baseline (speedup 1.0000x reference)
import functools

import jax
import jax.numpy as jnp
from jax import lax
from jax.experimental import pallas as pl
from jax.experimental.pallas import tpu as pltpu
from jax.experimental.pallas import tpu_sc as plsc

_B, _L = 16384, 50
_H = 64
_N = _B * _L
_NC, _NS = 2, 16
_NW = _NC * _NS
_ROWS_PER_TILE = _B // _NW
_NBUF = 4

_BB = 64
_TBLK = _BB * _L
_GRID = _B // _BB


_sc_mesh = plsc.VectorSubcoreMesh(core_axis_name="c", subcore_axis_name="s")


@functools.partial(
    pl.kernel,
    mesh=_sc_mesh,
    out_type=(jax.ShapeDtypeStruct((_N, _H), jnp.float32),
              jax.ShapeDtypeStruct((_N, _H), jnp.float32)),
    scratch_types=[
        pltpu.VMEM((_ROWS_PER_TILE, _L), jnp.int32),
        pltpu.VMEM((_ROWS_PER_TILE, _L), jnp.int32),
        pltpu.VMEM((_NBUF, _L, _H), jnp.float32),
        pltpu.VMEM((_NBUF, _L, _H), jnp.float32),
        pltpu.SemaphoreType.DMA,
        pltpu.SemaphoreType.DMA,
        pltpu.SemaphoreType.DMA,
        pltpu.SemaphoreType.DMA,
        pltpu.SemaphoreType.DMA,
        pltpu.SemaphoreType.DMA,
        pltpu.SemaphoreType.DMA,
        pltpu.SemaphoreType.DMA,
    ],
    compiler_params=pltpu.CompilerParams(use_tc_tiling_on_sc=False),
)
def _sc_gather(table_hbm, ptab_hbm, idx_hbm, pos_hbm, outT_hbm, outP_hbm,
               idx_v, pos_v, bufT, bufP, sg0, sg1, sg2, sg3, sw0, sw1, sw2,
               sw3):
    wid = lax.axis_index("s") * _NC + lax.axis_index("c")
    brow0 = wid * _ROWS_PER_TILE
    trow0 = brow0 * _L
    sgs = [sg0, sg1, sg2, sg3]
    sws = [sw0, sw1, sw2, sw3]
    pltpu.sync_copy(idx_hbm.at[pl.ds(brow0, _ROWS_PER_TILE)], idx_v)
    pltpu.sync_copy(pos_hbm.at[pl.ds(brow0, _ROWS_PER_TILE)], pos_v)

    def gathers(r, b, sem):
        pltpu.async_copy(table_hbm.at[idx_v.at[r]], bufT.at[b], sem)
        pltpu.async_copy(ptab_hbm.at[pos_v.at[r]], bufP.at[b], sem)

    def wait_gathers(r, b, sem):
        pltpu.make_async_copy(table_hbm.at[idx_v.at[r]], bufT.at[b], sem).wait()
        pltpu.make_async_copy(ptab_hbm.at[pos_v.at[r]], bufP.at[b], sem).wait()

    def writebacks(r, b, sem):
        pltpu.async_copy(bufT.at[b], outT_hbm.at[pl.ds(trow0 + r * _L, _L)],
                         sem)
        pltpu.async_copy(bufP.at[b], outP_hbm.at[pl.ds(trow0 + r * _L, _L)],
                         sem)

    def wait_writebacks(r, b, sem):
        pltpu.make_async_copy(
            bufT.at[b], outT_hbm.at[pl.ds(trow0 + r * _L, _L)], sem).wait()
        pltpu.make_async_copy(
            bufP.at[b], outP_hbm.at[pl.ds(trow0 + r * _L, _L)], sem).wait()

    for b in range(_NBUF):
        gathers(b, b, sgs[b])

    def body(i, carry):
        r = i * _NBUF
        for b in range(_NBUF):
            wait_gathers(r + b, b, sgs[b])
            writebacks(r + b, b, sws[b])
            nxt = r + b + _NBUF

            @pl.when(nxt < _ROWS_PER_TILE)
            def _():
                wait_writebacks(r + b, b, sws[b])
                gathers(nxt, b, sgs[b])

        return carry

    lax.fori_loop(0, _ROWS_PER_TILE // _NBUF, body, 0)
    for b in range(_NBUF):
        wait_writebacks(_ROWS_PER_TILE - _NBUF + b, b, sws[b])


def _tc_body(rep_ref, pose_ref, w2_ref, b2_ref, out_ref):
    x = rep_ref[...] + pose_ref[...]
    mean = jnp.mean(x, axis=1, keepdims=True)
    xc = x - mean
    var = jnp.mean(xc * xc, axis=1, keepdims=True)
    xn = xc * lax.rsqrt(var + 1e-5)
    y = jnp.dot(xn, w2_ref[...], preferred_element_type=jnp.float32) \
        + b2_ref[...]
    out_ref[...] = y.reshape(_BB, _L, _H)


_tc_call = pl.pallas_call(
    _tc_body,
    grid=(_GRID,),
    in_specs=[
        pl.BlockSpec((_TBLK, _H), lambda i: (i, 0)),
        pl.BlockSpec((_TBLK, _H), lambda i: (i, 0)),
        pl.BlockSpec((_H, _H), lambda i: (0, 0)),
        pl.BlockSpec((1, _H), lambda i: (0, 0)),
    ],
    out_specs=pl.BlockSpec((_BB, _L, _H), lambda i: (i, 0, 0)),
    out_shape=jax.ShapeDtypeStruct((_B, _L, _H), jnp.float32),
    compiler_params=pltpu.CompilerParams(
        dimension_semantics=("arbitrary",)),
)


def kernel(input, pos, token_table, pos_table, gamma, beta, W, b):
    rep, pose = _sc_gather(token_table, pos_table, input, pos)
    w2 = gamma[:, None] * W.T
    b2 = (beta @ W.T + b).reshape(1, _H)
    return _tc_call(rep, pose, w2, b2)


# speedup vs baseline: 2.0183x; 2.0183x over previous
import functools

import jax
import jax.numpy as jnp
from jax import lax
from jax.experimental import pallas as pl
from jax.experimental.pallas import tpu as pltpu
from jax.experimental.pallas import tpu_sc as plsc

_B, _L = 16384, 50
_H = 64
_N = _B * _L
_NC, _NS = 2, 16
_NW = _NC * _NS
_ROWS_PER_TILE = _B // _NW
_NBUF = 4

_BB = 64
_TBLK = _BB * _L
_GRID = _B // _BB


_sc_mesh = plsc.VectorSubcoreMesh(core_axis_name="c", subcore_axis_name="s")


@functools.partial(
    pl.kernel,
    mesh=_sc_mesh,
    out_type=jax.ShapeDtypeStruct((_N, 2 * _H), jnp.float32),
    scratch_types=[
        pltpu.VMEM((_ROWS_PER_TILE, _L), jnp.int32),
        pltpu.VMEM((_NBUF, _L, 2 * _H), jnp.float32),
        pltpu.SemaphoreType.DMA,
        pltpu.SemaphoreType.DMA,
        pltpu.SemaphoreType.DMA,
        pltpu.SemaphoreType.DMA,
        pltpu.SemaphoreType.DMA,
        pltpu.SemaphoreType.DMA,
        pltpu.SemaphoreType.DMA,
        pltpu.SemaphoreType.DMA,
    ],
    compiler_params=pltpu.CompilerParams(use_tc_tiling_on_sc=False),
)
def _sc_gather(table_hbm, idx_hbm, out_hbm, idx_v, bufs, sg0, sg1, sg2, sg3,
               sw0, sw1, sw2, sw3):
    wid = lax.axis_index("s") * _NC + lax.axis_index("c")
    brow0 = wid * _ROWS_PER_TILE
    trow0 = brow0 * _L
    sgs = [sg0, sg1, sg2, sg3]
    sws = [sw0, sw1, sw2, sw3]
    pltpu.sync_copy(idx_hbm.at[pl.ds(brow0, _ROWS_PER_TILE)], idx_v)

    for b in range(_NBUF):
        pltpu.async_copy(table_hbm.at[idx_v.at[b]], bufs.at[b], sgs[b])

    def body(i, carry):
        r = i * _NBUF
        for b in range(_NBUF):
            pltpu.make_async_copy(
                table_hbm.at[idx_v.at[r + b]], bufs.at[b], sgs[b]).wait()
            pltpu.async_copy(
                bufs.at[b], out_hbm.at[pl.ds(trow0 + (r + b) * _L, _L)],
                sws[b])
            nxt = r + b + _NBUF

            @pl.when(nxt < _ROWS_PER_TILE)
            def _():
                pltpu.make_async_copy(
                    bufs.at[b],
                    out_hbm.at[pl.ds(trow0 + (r + b) * _L, _L)],
                    sws[b]).wait()
                pltpu.async_copy(table_hbm.at[idx_v.at[nxt]], bufs.at[b],
                                 sgs[b])

        return carry

    lax.fori_loop(0, _ROWS_PER_TILE // _NBUF, body, 0)
    for b in range(_NBUF):
        last = _ROWS_PER_TILE - _NBUF + b
        pltpu.make_async_copy(
            bufs.at[b], out_hbm.at[pl.ds(trow0 + last * _L, _L)],
            sws[b]).wait()


def _tc_body(rep_ref, pos_ref, ptab_ref, w2_ref, b2_ref, out_ref):
    x = rep_ref[:, : _H]
    p = pos_ref[...].reshape(_TBLK, 1)
    onehot = (p == lax.broadcasted_iota(jnp.int32, (_TBLK, _H), 1))
    x = x + jnp.dot(onehot.astype(jnp.float32), ptab_ref[...],
                    preferred_element_type=jnp.float32)
    mean = jnp.mean(x, axis=1, keepdims=True)
    xc = x - mean
    var = jnp.mean(xc * xc, axis=1, keepdims=True)
    xn = xc * lax.rsqrt(var + 1e-5)
    y = jnp.dot(xn, w2_ref[...], preferred_element_type=jnp.float32) \
        + b2_ref[...]
    out_ref[...] = y.reshape(_BB, _L, _H)


_tc_call = pl.pallas_call(
    _tc_body,
    grid=(_GRID,),
    in_specs=[
        pl.BlockSpec((_TBLK, 2 * _H), lambda i: (i, 0)),
        pl.BlockSpec((1, 1, _TBLK), lambda i: (i, 0, 0)),
        pl.BlockSpec((_H, _H), lambda i: (0, 0)),
        pl.BlockSpec((_H, _H), lambda i: (0, 0)),
        pl.BlockSpec((1, _H), lambda i: (0, 0)),
    ],
    out_specs=pl.BlockSpec((_BB, _L, _H), lambda i: (i, 0, 0)),
    out_shape=jax.ShapeDtypeStruct((_B, _L, _H), jnp.float32),
    compiler_params=pltpu.CompilerParams(
        dimension_semantics=("arbitrary",)),
)


def kernel(input, pos, token_table, pos_table, gamma, beta, W, b):
    tpad = jnp.pad(token_table, ((0, 0), (0, _H)))
    rep = _sc_gather(tpad, input)
    posr = pos.reshape(_GRID, 1, _TBLK)
    ptab = jnp.zeros((_H, _H), jnp.float32).at[:pos_table.shape[0]].set(pos_table)
    w2 = gamma[:, None] * W.T
    b2 = (beta @ W.T + b).reshape(1, _H)
    return _tc_call(rep, posr, ptab, w2, b2)


# speedup vs baseline: 2.2720x; 1.1257x over previous
import functools

import jax
import jax.numpy as jnp
from jax import lax
from jax.experimental import pallas as pl
from jax.experimental.pallas import tpu as pltpu
from jax.experimental.pallas import tpu_sc as plsc

_B, _L = 16384, 50
_H = 64
_N = _B * _L
_NC, _NS = 2, 16
_NW = _NC * _NS
_ROWS_PER_TILE = _B // _NW
_NBUF = 4

_BB = 64
_TBLK = _BB * _L
_GRID = _B // _BB


_sc_mesh = plsc.VectorSubcoreMesh(core_axis_name="c", subcore_axis_name="s")


@functools.partial(
    pl.kernel,
    mesh=_sc_mesh,
    out_type=jax.ShapeDtypeStruct((_N, 2 * _H), jnp.float32),
    scratch_types=[
        pltpu.VMEM((_ROWS_PER_TILE, _L), jnp.int32),
        pltpu.VMEM((_NBUF, _L, 2 * _H), jnp.float32),
        pltpu.SemaphoreType.DMA,
        pltpu.SemaphoreType.DMA,
        pltpu.SemaphoreType.DMA,
        pltpu.SemaphoreType.DMA,
        pltpu.SemaphoreType.DMA,
        pltpu.SemaphoreType.DMA,
        pltpu.SemaphoreType.DMA,
        pltpu.SemaphoreType.DMA,
    ],
    compiler_params=pltpu.CompilerParams(use_tc_tiling_on_sc=False),
)
def _sc_gather(table_hbm, idx_hbm, out_hbm, idx_v, bufs, sg0, sg1, sg2, sg3,
               sw0, sw1, sw2, sw3):
    wid = lax.axis_index("s") * _NC + lax.axis_index("c")
    brow0 = wid * _ROWS_PER_TILE
    trow0 = brow0 * _L
    sgs = [sg0, sg1, sg2, sg3]
    sws = [sw0, sw1, sw2, sw3]
    pltpu.sync_copy(idx_hbm.at[pl.ds(brow0, _ROWS_PER_TILE)], idx_v)

    for b in range(_NBUF):
        pltpu.async_copy(table_hbm.at[idx_v.at[b]], bufs.at[b], sgs[b])

    def body(i, carry):
        r = i * _NBUF
        for b in range(_NBUF):
            pltpu.make_async_copy(
                table_hbm.at[idx_v.at[r + b]], bufs.at[b], sgs[b]).wait()
            pltpu.async_copy(
                bufs.at[b], out_hbm.at[pl.ds(trow0 + (r + b) * _L, _L)],
                sws[b])
            nxt = r + b + _NBUF

            @pl.when(nxt < _ROWS_PER_TILE)
            def _():
                pltpu.make_async_copy(
                    bufs.at[b],
                    out_hbm.at[pl.ds(trow0 + (r + b) * _L, _L)],
                    sws[b]).wait()
                pltpu.async_copy(table_hbm.at[idx_v.at[nxt]], bufs.at[b],
                                 sgs[b])

        return carry

    lax.fori_loop(0, _ROWS_PER_TILE // _NBUF, body, 0)
    for b in range(_NBUF):
        last = _ROWS_PER_TILE - _NBUF + b
        pltpu.make_async_copy(
            bufs.at[b], out_hbm.at[pl.ds(trow0 + last * _L, _L)],
            sws[b]).wait()


_BKI = 4096


def _pad_body(ttT_ref, out_ref):
    out_ref[:, : _H] = ttT_ref[...].T


_pad_call = pl.pallas_call(
    _pad_body,
    grid=(1000000 // _BKI,),
    in_specs=[pl.BlockSpec((_H, _BKI), lambda i: (0, i))],
    out_specs=pl.BlockSpec((_BKI, 2 * _H), lambda i: (i, 0)),
    out_shape=jax.ShapeDtypeStruct((1000000, 2 * _H), jnp.float32),
    compiler_params=pltpu.CompilerParams(
        dimension_semantics=("arbitrary",)),
)


def _tc_body(rep_ref, pos_ref, ptab_ref, w2_ref, b2_ref, out_ref):
    x = rep_ref[:, : _H]
    p = pos_ref[...].reshape(_TBLK, 1)
    onehot = (p == lax.broadcasted_iota(jnp.int32, (_TBLK, _H), 1))
    x = x + jnp.dot(onehot.astype(jnp.float32), ptab_ref[...],
                    preferred_element_type=jnp.float32)
    mean = jnp.mean(x, axis=1, keepdims=True)
    xc = x - mean
    var = jnp.mean(xc * xc, axis=1, keepdims=True)
    xn = xc * lax.rsqrt(var + 1e-5)
    y = jnp.dot(xn, w2_ref[...], preferred_element_type=jnp.float32) \
        + b2_ref[...]
    out_ref[...] = y.reshape(_BB, _L, _H)


_tc_call = pl.pallas_call(
    _tc_body,
    grid=(_GRID,),
    in_specs=[
        pl.BlockSpec((_TBLK, 2 * _H), lambda i: (i, 0)),
        pl.BlockSpec((1, 1, _TBLK), lambda i: (i, 0, 0)),
        pl.BlockSpec((_H, _H), lambda i: (0, 0)),
        pl.BlockSpec((_H, _H), lambda i: (0, 0)),
        pl.BlockSpec((1, _H), lambda i: (0, 0)),
    ],
    out_specs=pl.BlockSpec((_BB, _L, _H), lambda i: (i, 0, 0)),
    out_shape=jax.ShapeDtypeStruct((_B, _L, _H), jnp.float32),
    compiler_params=pltpu.CompilerParams(
        dimension_semantics=("arbitrary",)),
)


def kernel(input, pos, token_table, pos_table, gamma, beta, W, b):
    tpad = _pad_call(token_table.T)
    rep = _sc_gather(tpad, input)
    posr = pos.reshape(_GRID, 1, _TBLK)
    ptab = jnp.zeros((_H, _H), jnp.float32).at[:pos_table.shape[0]].set(pos_table)
    w2 = gamma[:, None] * W.T
    b2 = (beta @ W.T + b).reshape(1, _H)
    return _tc_call(rep, posr, ptab, w2, b2)


# speedup vs baseline: 2.2754x; 1.0015x over previous
import functools

import jax
import jax.numpy as jnp
from jax import lax
from jax.experimental import pallas as pl
from jax.experimental.pallas import tpu as pltpu
from jax.experimental.pallas import tpu_sc as plsc

_B, _L = 16384, 50
_H = 64
_N = _B * _L
_NC, _NS = 2, 16
_NW = _NC * _NS
_ROWS_PER_TILE = _B // _NW
_NBUF = 4

_BB = 64
_TBLK = _BB * _L
_GRID = _B // _BB


_sc_mesh = plsc.VectorSubcoreMesh(core_axis_name="c", subcore_axis_name="s")


@functools.partial(
    pl.kernel,
    mesh=_sc_mesh,
    out_type=jax.ShapeDtypeStruct((_N, 2 * _H), jnp.float32),
    scratch_types=[
        pltpu.VMEM((_ROWS_PER_TILE, _L), jnp.int32),
        pltpu.VMEM((_NBUF, _L, 2 * _H), jnp.float32),
        pltpu.SemaphoreType.DMA,
        pltpu.SemaphoreType.DMA,
        pltpu.SemaphoreType.DMA,
        pltpu.SemaphoreType.DMA,
        pltpu.SemaphoreType.DMA,
        pltpu.SemaphoreType.DMA,
        pltpu.SemaphoreType.DMA,
        pltpu.SemaphoreType.DMA,
    ],
    compiler_params=pltpu.CompilerParams(use_tc_tiling_on_sc=False),
)
def _sc_gather(table_hbm, idx_hbm, out_hbm, idx_v, bufs, sg0, sg1, sg2, sg3,
               sw0, sw1, sw2, sw3):
    wid = lax.axis_index("s") * _NC + lax.axis_index("c")
    brow0 = wid * _ROWS_PER_TILE
    trow0 = brow0 * _L
    sgs = [sg0, sg1, sg2, sg3]
    sws = [sw0, sw1, sw2, sw3]
    pltpu.sync_copy(idx_hbm.at[pl.ds(brow0, _ROWS_PER_TILE)], idx_v)

    for b in range(_NBUF):
        pltpu.async_copy(table_hbm.at[idx_v.at[b]], bufs.at[b], sgs[b])

    def body(i, carry):
        r = i * _NBUF
        for b in range(_NBUF):
            pltpu.make_async_copy(
                table_hbm.at[idx_v.at[r + b]], bufs.at[b], sgs[b]).wait()
            pltpu.async_copy(
                bufs.at[b], out_hbm.at[pl.ds(trow0 + (r + b) * _L, _L)],
                sws[b])
            nxt = r + b + _NBUF

            @pl.when(nxt < _ROWS_PER_TILE)
            def _():
                pltpu.make_async_copy(
                    bufs.at[b],
                    out_hbm.at[pl.ds(trow0 + (r + b) * _L, _L)],
                    sws[b]).wait()
                pltpu.async_copy(table_hbm.at[idx_v.at[nxt]], bufs.at[b],
                                 sgs[b])

        return carry

    lax.fori_loop(0, _ROWS_PER_TILE // _NBUF, body, 0)
    for b in range(_NBUF):
        last = _ROWS_PER_TILE - _NBUF + b
        pltpu.make_async_copy(
            bufs.at[b], out_hbm.at[pl.ds(trow0 + last * _L, _L)],
            sws[b]).wait()


_BKI = 4096


def _pad_body(ttT_ref, out_ref):
    out_ref[:, : _H] = ttT_ref[...].T


_pad_call = pl.pallas_call(
    _pad_body,
    grid=((1000000 + _BKI - 1) // _BKI,),
    in_specs=[pl.BlockSpec((_H, _BKI), lambda i: (0, i))],
    out_specs=pl.BlockSpec((_BKI, 2 * _H), lambda i: (i, 0)),
    out_shape=jax.ShapeDtypeStruct((1000000, 2 * _H), jnp.float32),
    compiler_params=pltpu.CompilerParams(
        dimension_semantics=("arbitrary",)),
)


def _tc_body(rep_ref, pos_ref, ptab_ref, w2_ref, b2_ref, out_ref):
    x = rep_ref[:, : _H]
    p = pos_ref[...].reshape(_TBLK, 1)
    onehot = (p == lax.broadcasted_iota(jnp.int32, (_TBLK, _H), 1))
    x = x + jnp.dot(onehot.astype(jnp.float32), ptab_ref[...],
                    preferred_element_type=jnp.float32)
    mean = jnp.mean(x, axis=1, keepdims=True)
    xc = x - mean
    var = jnp.mean(xc * xc, axis=1, keepdims=True)
    xn = xc * lax.rsqrt(var + 1e-5)
    y = jnp.dot(xn, w2_ref[...], preferred_element_type=jnp.float32) \
        + b2_ref[...]
    out_ref[...] = y.reshape(_BB, _L, _H)


_tc_call = pl.pallas_call(
    _tc_body,
    grid=(_GRID,),
    in_specs=[
        pl.BlockSpec((_TBLK, 2 * _H), lambda i: (i, 0)),
        pl.BlockSpec((1, 1, _TBLK), lambda i: (i, 0, 0)),
        pl.BlockSpec((_H, _H), lambda i: (0, 0)),
        pl.BlockSpec((_H, _H), lambda i: (0, 0)),
        pl.BlockSpec((1, _H), lambda i: (0, 0)),
    ],
    out_specs=pl.BlockSpec((_BB, _L, _H), lambda i: (i, 0, 0)),
    out_shape=jax.ShapeDtypeStruct((_B, _L, _H), jnp.float32),
    compiler_params=pltpu.CompilerParams(
        dimension_semantics=("arbitrary",)),
)


def kernel(input, pos, token_table, pos_table, gamma, beta, W, b):
    tpad = _pad_call(token_table.T)
    rep = _sc_gather(tpad, input)
    posr = pos.reshape(_GRID, 1, _TBLK)
    ptab = jnp.zeros((_H, _H), jnp.float32).at[:pos_table.shape[0]].set(pos_table)
    w2 = gamma[:, None] * W.T
    b2 = (beta @ W.T + b).reshape(1, _H)
    return _tc_call(rep, posr, ptab, w2, b2)


# speedup vs baseline: 2.3379x; 1.0275x over previous
import functools

import jax
import jax.numpy as jnp
from jax import lax
from jax.experimental import pallas as pl
from jax.experimental.pallas import tpu as pltpu
from jax.experimental.pallas import tpu_sc as plsc

_B, _L = 16384, 50
_H = 64
_N = _B * _L
_NC, _NS = 2, 16
_NW = _NC * _NS
_ROWS_PER_TILE = _B // _NW
_NBUF = 4

_BB = 128
_TBLK = _BB * _L
_GRID = _B // _BB


_sc_mesh = plsc.VectorSubcoreMesh(core_axis_name="c", subcore_axis_name="s")


@functools.partial(
    pl.kernel,
    mesh=_sc_mesh,
    out_type=jax.ShapeDtypeStruct((_N, 2 * _H), jnp.float32),
    scratch_types=[
        pltpu.VMEM((_ROWS_PER_TILE, _L), jnp.int32),
        pltpu.VMEM((_NBUF, _L, 2 * _H), jnp.float32),
        pltpu.SemaphoreType.DMA,
        pltpu.SemaphoreType.DMA,
        pltpu.SemaphoreType.DMA,
        pltpu.SemaphoreType.DMA,
        pltpu.SemaphoreType.DMA,
        pltpu.SemaphoreType.DMA,
        pltpu.SemaphoreType.DMA,
        pltpu.SemaphoreType.DMA,
    ],
    compiler_params=pltpu.CompilerParams(use_tc_tiling_on_sc=False),
)
def _sc_gather(table_hbm, idx_hbm, out_hbm, idx_v, bufs, sg0, sg1, sg2, sg3,
               sw0, sw1, sw2, sw3):
    wid = lax.axis_index("s") * _NC + lax.axis_index("c")
    brow0 = wid * _ROWS_PER_TILE
    trow0 = brow0 * _L
    sgs = [sg0, sg1, sg2, sg3]
    sws = [sw0, sw1, sw2, sw3]
    pltpu.sync_copy(idx_hbm.at[pl.ds(brow0, _ROWS_PER_TILE)], idx_v)

    for b in range(_NBUF):
        pltpu.async_copy(table_hbm.at[idx_v.at[b]], bufs.at[b], sgs[b])

    def body(i, carry):
        r = i * _NBUF
        for b in range(_NBUF):
            pltpu.make_async_copy(
                table_hbm.at[idx_v.at[r + b]], bufs.at[b], sgs[b]).wait()
            pltpu.async_copy(
                bufs.at[b], out_hbm.at[pl.ds(trow0 + (r + b) * _L, _L)],
                sws[b])
            nxt = r + b + _NBUF

            @pl.when(nxt < _ROWS_PER_TILE)
            def _():
                pltpu.make_async_copy(
                    bufs.at[b],
                    out_hbm.at[pl.ds(trow0 + (r + b) * _L, _L)],
                    sws[b]).wait()
                pltpu.async_copy(table_hbm.at[idx_v.at[nxt]], bufs.at[b],
                                 sgs[b])

        return carry

    lax.fori_loop(0, _ROWS_PER_TILE // _NBUF, body, 0)
    for b in range(_NBUF):
        last = _ROWS_PER_TILE - _NBUF + b
        pltpu.make_async_copy(
            bufs.at[b], out_hbm.at[pl.ds(trow0 + last * _L, _L)],
            sws[b]).wait()


_BKI = 4096


def _pad_body(ttT_ref, out_ref):
    out_ref[:, : _H] = ttT_ref[...].T


_pad_call = pl.pallas_call(
    _pad_body,
    grid=((1000000 + _BKI - 1) // _BKI,),
    in_specs=[pl.BlockSpec((_H, _BKI), lambda i: (0, i))],
    out_specs=pl.BlockSpec((_BKI, 2 * _H), lambda i: (i, 0)),
    out_shape=jax.ShapeDtypeStruct((1000000, 2 * _H), jnp.float32),
    compiler_params=pltpu.CompilerParams(
        dimension_semantics=("arbitrary",)),
)


def _tc_body(rep_ref, pos_ref, ptab_ref, w2_ref, b2_ref, out_ref):
    x = rep_ref[:, : _H]
    p = pos_ref[...].reshape(_TBLK, 1)
    onehot = (p == lax.broadcasted_iota(jnp.int32, (_TBLK, _H), 1))
    x = x + jnp.dot(onehot.astype(jnp.float32), ptab_ref[...],
                    preferred_element_type=jnp.float32)
    mean = jnp.mean(x, axis=1, keepdims=True)
    xc = x - mean
    var = jnp.mean(xc * xc, axis=1, keepdims=True)
    xn = xc * lax.rsqrt(var + 1e-5)
    y = jnp.dot(xn, w2_ref[...], preferred_element_type=jnp.float32) \
        + b2_ref[...]
    out_ref[...] = y.reshape(_BB, _L, _H)


_tc_call = pl.pallas_call(
    _tc_body,
    grid=(_GRID,),
    in_specs=[
        pl.BlockSpec((_TBLK, 2 * _H), lambda i: (i, 0)),
        pl.BlockSpec((1, 1, _TBLK), lambda i: (i, 0, 0)),
        pl.BlockSpec((_H, _H), lambda i: (0, 0)),
        pl.BlockSpec((_H, _H), lambda i: (0, 0)),
        pl.BlockSpec((1, _H), lambda i: (0, 0)),
    ],
    out_specs=pl.BlockSpec((_BB, _L, _H), lambda i: (i, 0, 0)),
    out_shape=jax.ShapeDtypeStruct((_B, _L, _H), jnp.float32),
    compiler_params=pltpu.CompilerParams(
        dimension_semantics=("arbitrary",)),
)


def kernel(input, pos, token_table, pos_table, gamma, beta, W, b):
    tpad = _pad_call(token_table.T)
    rep = _sc_gather(tpad, input)
    posr = pos.reshape(_GRID, 1, _TBLK)
    ptab = jnp.zeros((_H, _H), jnp.float32).at[:pos_table.shape[0]].set(pos_table)
    w2 = gamma[:, None] * W.T
    b2 = (beta @ W.T + b).reshape(1, _H)
    return _tc_call(rep, posr, ptab, w2, b2)
